# baseline (speedup 1.0000x reference)
import functools

import jax
import jax.numpy as jnp
from jax.experimental import pallas as pl
from jax.experimental.pallas import tpu as pltpu

_GRID = (32, 32)
_LATENT = 256
_ALPHA = 1.0
_N_NODES = _GRID[0] * _GRID[1]
_BLK = 256


def _som_block(z_ref, tw_ref, nodes_t_ref, nodes_ref, som_ref, q_ref, idx_ref):
    z = z_ref[...]
    tw = tw_ref[...]
    nodes_t = nodes_t_ref[...]
    wz = z * tw

    mm = jnp.dot(wz, nodes_t, preferred_element_type=jnp.float32)
    nn = jnp.sum(nodes_t * nodes_t, axis=0, keepdims=True)
    rowsq = jnp.sum(wz * wz, axis=1, keepdims=True)
    sq = rowsq - 2.0 * mm + nn
    dists = jnp.sqrt(jnp.maximum(sq, 1e-12))

    q_raw = 1.0 / (1.0 + dists / _ALPHA)
    q_ref[...] = q_raw / jnp.sum(q_raw, axis=1, keepdims=True)

    idx = jnp.argmin(sq, axis=1).astype(jnp.int32)
    idx_col = idx[:, None]
    idx_ref[...] = idx_col

    lane = jax.lax.broadcasted_iota(jnp.int32, dists.shape, 1)
    onehot = (lane == idx_col).astype(jnp.float32)
    gathered = jnp.dot(onehot, nodes_ref[...],
                       preferred_element_type=jnp.float32)
    som_ref[...] = 0.9 * z + 0.1 * gathered


@jax.jit
def kernel(z, nodes, time_weights):
    b, t, d = z.shape
    n_rows = b * t
    z_flat = z.reshape(n_rows, d)
    nodes_flat = nodes.reshape(-1, d)
    nodes_t = nodes_flat.T
    tw_col = time_weights[0, -t:, :]

    n_blocks = n_rows // _BLK
    tw_blocks = t // _BLK if t >= _BLK else 1

    som, q, idx = pl.pallas_call(
        _som_block,
        grid=(n_blocks,),
        in_specs=[
            pl.BlockSpec((_BLK, d), lambda i: (i, 0)),
            pl.BlockSpec((_BLK, 1), lambda i: (i % tw_blocks, 0)),
            pl.BlockSpec((d, _N_NODES), lambda i: (0, 0)),
            pl.BlockSpec((_N_NODES, d), lambda i: (0, 0)),
        ],
        out_specs=[
            pl.BlockSpec((_BLK, d), lambda i: (i, 0)),
            pl.BlockSpec((_BLK, _N_NODES), lambda i: (i, 0)),
            pl.BlockSpec((_BLK, 1), lambda i: (i, 0)),
        ],
        out_shape=[
            jax.ShapeDtypeStruct((n_rows, d), jnp.float32),
            jax.ShapeDtypeStruct((n_rows, _N_NODES), jnp.float32),
            jax.ShapeDtypeStruct((n_rows, 1), jnp.int32),
        ],
    )(z_flat, tw_col, nodes_t, nodes_flat)

    som_z = som.reshape(b, t, d)
    bmu_indices = idx[:, 0].reshape(b, t)
    return som_z, q, bmu_indices


# speedup vs baseline: 1.0507x; 1.0507x over previous
import functools

import jax
import jax.numpy as jnp
from jax.experimental import pallas as pl
from jax.experimental.pallas import tpu as pltpu

_GRID = (32, 32)
_LATENT = 256
_ALPHA = 1.0
_N_NODES = _GRID[0] * _GRID[1]
_BLK = 512


def _som_block(z_ref, tw_ref, nodes_t_ref, nodes_ref, som_ref, q_ref, idx_ref):
    z = z_ref[...]
    tw = tw_ref[...]
    nodes_t = nodes_t_ref[...]
    wz = z * tw

    mm = jnp.dot(wz, nodes_t, preferred_element_type=jnp.float32)
    nn = jnp.sum(nodes_t * nodes_t, axis=0, keepdims=True)
    rowsq = jnp.sum(wz * wz, axis=1, keepdims=True)
    sq = rowsq - 2.0 * mm + nn
    dists = jnp.sqrt(jnp.maximum(sq, 1e-12))

    q_raw = 1.0 / (1.0 + dists / _ALPHA)
    q_ref[...] = q_raw / jnp.sum(q_raw, axis=1, keepdims=True)

    idx = jnp.argmin(sq, axis=1).astype(jnp.int32)
    idx_col = idx[:, None]
    idx_ref[...] = idx_col

    lane = jax.lax.broadcasted_iota(jnp.int32, dists.shape, 1)
    onehot = (lane == idx_col).astype(jnp.float32)
    gathered = jnp.dot(onehot, nodes_ref[...],
                       preferred_element_type=jnp.float32)
    som_ref[...] = 0.9 * z + 0.1 * gathered


@jax.jit
def kernel(z, nodes, time_weights):
    b, t, d = z.shape
    n_rows = b * t
    z_flat = z.reshape(n_rows, d)
    nodes_flat = nodes.reshape(-1, d)
    nodes_t = nodes_flat.T
    tw_col = time_weights[0, -t:, :]

    n_blocks = n_rows // _BLK
    tw_blocks = t // _BLK if t >= _BLK else 1

    som, q, idx = pl.pallas_call(
        _som_block,
        grid=(n_blocks,),
        in_specs=[
            pl.BlockSpec((_BLK, d), lambda i: (i, 0)),
            pl.BlockSpec((_BLK, 1), lambda i: (i % tw_blocks, 0)),
            pl.BlockSpec((d, _N_NODES), lambda i: (0, 0)),
            pl.BlockSpec((_N_NODES, d), lambda i: (0, 0)),
        ],
        out_specs=[
            pl.BlockSpec((_BLK, d), lambda i: (i, 0)),
            pl.BlockSpec((_BLK, _N_NODES), lambda i: (i, 0)),
            pl.BlockSpec((_BLK, 1), lambda i: (i, 0)),
        ],
        out_shape=[
            jax.ShapeDtypeStruct((n_rows, d), jnp.float32),
            jax.ShapeDtypeStruct((n_rows, _N_NODES), jnp.float32),
            jax.ShapeDtypeStruct((n_rows, 1), jnp.int32),
        ],
    )(z_flat, tw_col, nodes_t, nodes_flat)

    som_z = som.reshape(b, t, d)
    bmu_indices = idx[:, 0].reshape(b, t)
    return som_z, q, bmu_indices


# speedup vs baseline: 1.0664x; 1.0150x over previous
import jax
import jax.numpy as jnp
from jax.experimental import pallas as pl

_GRID = (32, 32)
_ALPHA = 1.0
_N_NODES = _GRID[0] * _GRID[1]
_BLK = 512

_DN_T = (((1,), (1,)), ((), ()))


def _som_block(z_ref, tw_ref, nodes_ref, som_ref, q_ref, idx_ref):
    z = z_ref[...]
    tw = tw_ref[...]
    nodes = nodes_ref[...]
    wz = z * tw

    mm = jax.lax.dot_general(wz, nodes, _DN_T,
                             preferred_element_type=jnp.float32)
    ones_row = jnp.ones((1, nodes.shape[1]), jnp.float32)
    nn = jax.lax.dot_general(ones_row, nodes * nodes, _DN_T,
                             preferred_element_type=jnp.float32)
    rowsq = jnp.sum(wz * wz, axis=1, keepdims=True)
    sq = rowsq - 2.0 * mm + nn
    dists = jnp.sqrt(jnp.maximum(sq, 1e-12))

    q_raw = 1.0 / (1.0 + dists / _ALPHA)
    q_ref[...] = q_raw / jnp.sum(q_raw, axis=1, keepdims=True)

    idx = jnp.argmin(sq, axis=1).astype(jnp.int32)
    idx_col = idx[:, None]
    idx_ref[...] = idx_col

    lane = jax.lax.broadcasted_iota(jnp.int32, sq.shape, 1)
    onehot = (lane == idx_col).astype(jnp.float32)
    gathered = jnp.dot(onehot, nodes,
                       preferred_element_type=jnp.float32)
    som_ref[...] = 0.9 * z + 0.1 * gathered


@jax.jit
def kernel(z, nodes, time_weights):
    b, t, d = z.shape
    n_rows = b * t
    z_flat = z.reshape(n_rows, d)
    nodes_flat = nodes.reshape(-1, d)
    tw_col = time_weights[0, -t:, :]

    n_blocks = n_rows // _BLK
    tw_blocks = t // _BLK if t >= _BLK else 1

    som, q, idx = pl.pallas_call(
        _som_block,
        grid=(n_blocks,),
        in_specs=[
            pl.BlockSpec((_BLK, d), lambda i: (i, 0)),
            pl.BlockSpec((_BLK, 1), lambda i: (i % tw_blocks, 0)),
            pl.BlockSpec((_N_NODES, d), lambda i: (0, 0)),
        ],
        out_specs=[
            pl.BlockSpec((_BLK, d), lambda i: (i, 0)),
            pl.BlockSpec((_BLK, _N_NODES), lambda i: (i, 0)),
            pl.BlockSpec((_BLK, 1), lambda i: (i, 0)),
        ],
        out_shape=[
            jax.ShapeDtypeStruct((n_rows, d), jnp.float32),
            jax.ShapeDtypeStruct((n_rows, _N_NODES), jnp.float32),
            jax.ShapeDtypeStruct((n_rows, 1), jnp.int32),
        ],
    )(z_flat, tw_col, nodes_flat)

    som_z = som.reshape(b, t, d)
    bmu_indices = idx[:, 0].reshape(b, t)
    return som_z, q, bmu_indices


# speedup vs baseline: 1.0748x; 1.0078x over previous
import jax
import jax.numpy as jnp
from jax.experimental import pallas as pl

_GRID = (32, 32)
_ALPHA = 1.0
_N_NODES = _GRID[0] * _GRID[1]
_BLK = 512


def _som_block(z_ref, tw_ref, nodes_t_ref, nodes_ref, som_ref, q_ref, idx_ref):
    z = z_ref[...]
    tw = tw_ref[...]
    nodes_t = nodes_t_ref[...]
    wz = z * tw

    mm = jnp.dot(wz, nodes_t, preferred_element_type=jnp.float32)
    nn = jnp.sum(nodes_t * nodes_t, axis=0, keepdims=True)
    rowsq = jnp.sum(wz * wz, axis=1, keepdims=True)
    sq = rowsq - 2.0 * mm + nn
    dists = jnp.sqrt(jnp.maximum(sq, 1e-12))

    q_raw = 1.0 / (1.0 + dists / _ALPHA)
    q_ref[...] = q_raw / jnp.sum(q_raw, axis=1, keepdims=True)

    idx = jnp.argmin(dists, axis=1).astype(jnp.int32)
    idx_col = idx[:, None]
    idx_ref[...] = idx_col

    lane = jax.lax.broadcasted_iota(jnp.int32, dists.shape, 1)
    onehot = (lane == idx_col).astype(jnp.float32)
    gathered = jnp.dot(onehot, nodes_ref[...],
                       preferred_element_type=jnp.float32)
    som_ref[...] = 0.9 * z + 0.1 * gathered


@jax.jit
def kernel(z, nodes, time_weights):
    b, t, d = z.shape
    n_rows = b * t
    z_flat = z.reshape(n_rows, d)
    nodes_flat = nodes.reshape(-1, d)
    nodes_t = nodes_flat.T
    tw_col = time_weights[0, -t:, :]

    n_blocks = n_rows // _BLK
    tw_blocks = t // _BLK if t >= _BLK else 1

    som, q, idx = pl.pallas_call(
        _som_block,
        grid=(n_blocks,),
        in_specs=[
            pl.BlockSpec((_BLK, d), lambda i: (i, 0)),
            pl.BlockSpec((_BLK, 1), lambda i: (i % tw_blocks, 0)),
            pl.BlockSpec((d, _N_NODES), lambda i: (0, 0)),
            pl.BlockSpec((_N_NODES, d), lambda i: (0, 0)),
        ],
        out_specs=[
            pl.BlockSpec((_BLK, d), lambda i: (i, 0)),
            pl.BlockSpec((_BLK, _N_NODES), lambda i: (i, 0)),
            pl.BlockSpec((_BLK, 1), lambda i: (i, 0)),
        ],
        out_shape=[
            jax.ShapeDtypeStruct((n_rows, d), jnp.float32),
            jax.ShapeDtypeStruct((n_rows, _N_NODES), jnp.float32),
            jax.ShapeDtypeStruct((n_rows, 1), jnp.int32),
        ],
    )(z_flat, tw_col, nodes_t, nodes_flat)

    som_z = som.reshape(b, t, d)
    bmu_indices = idx[:, 0].reshape(b, t)
    return som_z, q, bmu_indices


# speedup vs baseline: 1.1904x; 1.1076x over previous
import jax
import jax.numpy as jnp
from jax.experimental import pallas as pl
from jax.experimental.pallas import tpu as pltpu

_GRID = (32, 32)
_ALPHA = 1.0
_N_NODES = _GRID[0] * _GRID[1]
_BLK = 512

_DN_T = (((1,), (1,)), ((), ()))


def _som_block(z_ref, tw_ref, nodes_ref, som_ref, q_ref, idx_ref,
               nodes_t_ref, nn_ref):
    @pl.when(pl.program_id(0) == 0)
    def _prologue():
        nt = jnp.transpose(nodes_ref[...], (1, 0))
        nodes_t_ref[...] = nt
        nn_ref[...] = jnp.sum(nt * nt, axis=0, keepdims=True)

    z = z_ref[...]
    tw = tw_ref[...]
    nodes_t = nodes_t_ref[...]
    wz = z * tw

    mm = jnp.dot(wz, nodes_t, preferred_element_type=jnp.float32)
    rowsq = jnp.sum(wz * wz, axis=1, keepdims=True)
    sq = rowsq - 2.0 * mm + nn_ref[...]
    dists = jnp.sqrt(jnp.maximum(sq, 1e-12))

    q_raw = 1.0 / (1.0 + dists / _ALPHA)
    q_ref[...] = q_raw / jnp.sum(q_raw, axis=1, keepdims=True)

    idx = jnp.argmin(dists, axis=1).astype(jnp.int32)
    idx_col = idx[:, None]
    idx_ref[...] = idx_col

    lane = jax.lax.broadcasted_iota(jnp.int32, dists.shape, 1)
    onehot = (lane == idx_col).astype(jnp.float32)
    gathered = jax.lax.dot_general(onehot, nodes_t, _DN_T,
                                   preferred_element_type=jnp.float32)
    som_ref[...] = 0.9 * z + 0.1 * gathered


@jax.jit
def kernel(z, nodes, time_weights):
    b, t, d = z.shape
    n_rows = b * t
    z_flat = z.reshape(n_rows, d)
    nodes_flat = nodes.reshape(-1, d)
    tw_col = time_weights[0, -t:, :]

    n_blocks = n_rows // _BLK
    tw_blocks = t // _BLK if t >= _BLK else 1

    som, q, idx = pl.pallas_call(
        _som_block,
        grid=(n_blocks,),
        in_specs=[
            pl.BlockSpec((_BLK, d), lambda i: (i, 0)),
            pl.BlockSpec((_BLK, 1), lambda i: (i % tw_blocks, 0)),
            pl.BlockSpec((_N_NODES, d), lambda i: (0, 0)),
        ],
        out_specs=[
            pl.BlockSpec((_BLK, d), lambda i: (i, 0)),
            pl.BlockSpec((_BLK, _N_NODES), lambda i: (i, 0)),
            pl.BlockSpec((_BLK, 1), lambda i: (i, 0)),
        ],
        out_shape=[
            jax.ShapeDtypeStruct((n_rows, d), jnp.float32),
            jax.ShapeDtypeStruct((n_rows, _N_NODES), jnp.float32),
            jax.ShapeDtypeStruct((n_rows, 1), jnp.int32),
        ],
        scratch_shapes=[
            pltpu.VMEM((d, _N_NODES), jnp.float32),
            pltpu.VMEM((1, _N_NODES), jnp.float32),
        ],
    )(z_flat, tw_col, nodes_flat)

    som_z = som.reshape(b, t, d)
    bmu_indices = idx[:, 0].reshape(b, t)
    return som_z, q, bmu_indices
